# baseline (speedup 1.0000x reference)
import jax
import jax.numpy as jnp
from jax.experimental import pallas as pl

_LEVELS = 4
_CODES = 1024
_DIM = 64
_COMMIT = 0.25
_BLOCK = 256


def _rowsum(s):
    acc = s[:, 0:8]
    for j in range(1, 8):
        acc = acc + s[:, 8 * j:8 * j + 8]
    acc = acc[:, 0:4] + acc[:, 4:8]
    acc = acc[:, 0:2] + acc[:, 2:4]
    return acc[:, 0:1] + acc[:, 1:2]


def _rvq_kernel(x_ref, emb_ref, q_ref, idx_ref, loss_ref):
    i = pl.program_id(0)
    x = x_ref[...]
    res = x
    qs = jnp.zeros_like(x)
    level_sums = []
    level_idx = []
    for lvl in range(_LEVELS):
        emb = emb_ref[lvl]
        esq = _rowsum(emb * emb)[:, 0]
        rsq = _rowsum(res * res)
        prod = jax.lax.dot_general(
            res, emb, (((1,), (1,)), ((), ())),
            preferred_element_type=jnp.float32)
        d = rsq + esq[None, :] - 2.0 * prod
        dmin = jnp.min(d, axis=1, keepdims=True)
        lanes = jax.lax.broadcasted_iota(jnp.int32, d.shape, 1)
        idx = jnp.min(jnp.where(d == dmin, lanes, jnp.int32(2 ** 30)),
                      axis=1).astype(jnp.int32)
        onehot = (jax.lax.broadcasted_iota(jnp.int32, d.shape, 1)
                  == idx[:, None]).astype(jnp.float32)
        q = jax.lax.dot_general(
            onehot, emb, (((1,), (0,)), ((), ())),
            preferred_element_type=jnp.float32)
        res = res - q
        qs = qs + q
        level_sums.append(jnp.sum(res * res))
        level_idx.append(idx)
    q_ref[...] = x + (qs - x)
    idx_ref[...] = jnp.stack(level_idx, axis=0)
    vals = jnp.concatenate(
        [jnp.full((1, 128), s, jnp.float32) for s in level_sums], axis=0)

    @pl.when(i == 0)
    def _():
        loss_ref[...] = jnp.zeros_like(loss_ref)

    loss_ref[...] += vals


def kernel(inputs, embedding):
    n, dim = inputs.shape
    grid = n // _BLOCK
    out_shapes = (
        jax.ShapeDtypeStruct((n, dim), jnp.float32),
        jax.ShapeDtypeStruct((_LEVELS, n), jnp.int32),
        jax.ShapeDtypeStruct((_LEVELS, 128), jnp.float32),
    )
    q, idx, loss = pl.pallas_call(
        _rvq_kernel,
        grid=(grid,),
        in_specs=[
            pl.BlockSpec((_BLOCK, dim), lambda i: (i, 0)),
            pl.BlockSpec((_LEVELS, _CODES, _DIM), lambda i: (0, 0, 0)),
        ],
        out_specs=(
            pl.BlockSpec((_BLOCK, dim), lambda i: (i, 0)),
            pl.BlockSpec((_LEVELS, _BLOCK), lambda i: (0, i)),
            pl.BlockSpec((_LEVELS, 128), lambda i: (0, 0)),
        ),
        out_shape=out_shapes,
    )(inputs, embedding)
    denom = jnp.float32(n * dim)
    per_level = loss[:, 0] / denom
    cb = per_level[0] + per_level[1] + per_level[2] + per_level[3]
    commit = cb
    vq = cb + jnp.float32(_COMMIT) * commit
    return (q, idx, vq, cb, commit)


# speedup vs baseline: 3.0155x; 3.0155x over previous
import jax
import jax.numpy as jnp
from jax.experimental import pallas as pl
from jax.experimental.pallas import tpu as pltpu

_LEVELS = 4
_CODES = 1024
_DIM = 64
_COMMIT = 0.25
_BLOCK = 256
_BIG = 2 ** 30


def _foldsum(s):
    acc = s[0:8, :]
    for j in range(1, 8):
        acc = acc + s[8 * j:8 * j + 8, :]
    acc = acc[0:4, :] + acc[4:8, :]
    acc = acc[0:2, :] + acc[2:4, :]
    return acc[0:1, :] + acc[1:2, :]


def _rvq_kernel(x_ref, emb_ref, embT_ref, q_ref, idx_ref, loss_ref,
                esqb_ref, iota_ref):
    i = pl.program_id(0)

    @pl.when(i == 0)
    def _init():
        for lvl in range(_LEVELS):
            eT = embT_ref[lvl]
            esq_row = _foldsum(eT * eT)
            esqb_ref[lvl] = jnp.broadcast_to(
                esq_row.reshape(_CODES, 1), (_CODES, _BLOCK))
        iota_ref[...] = jax.lax.broadcasted_iota(
            jnp.int32, (_CODES, _BLOCK), 0)
        loss_ref[...] = jnp.zeros_like(loss_ref)

    xT = x_ref[...].T
    rowids = iota_ref[...]
    res = xT
    qs = jnp.zeros_like(xT)
    rsq = _foldsum(res * res)
    level_idx = []
    level_loss = []
    for lvl in range(_LEVELS):
        emb = emb_ref[lvl]
        prodT = jax.lax.dot_general(
            emb, res, (((1,), (0,)), ((), ())),
            preferred_element_type=jnp.float32)
        d = (esqb_ref[lvl] + rsq) - 2.0 * prodT
        dmin = jnp.min(d, axis=0, keepdims=True)
        idx = jnp.min(jnp.where(d == dmin, rowids, jnp.int32(_BIG)),
                      axis=0, keepdims=True)
        onehot = jnp.where(rowids == idx, jnp.float32(1.0),
                           jnp.float32(0.0))
        qT = jax.lax.dot_general(
            embT_ref[lvl], onehot, (((1,), (0,)), ((), ())),
            preferred_element_type=jnp.float32)
        res = res - qT
        qs = qs + qT
        rsq = _foldsum(res * res)
        level_idx.append(idx)
        level_loss.append(rsq)
    q_ref[...] = (xT + (qs - xT)).T
    idx_ref[...] = jnp.concatenate(level_idx, axis=0)
    loss_ref[...] += jnp.concatenate(level_loss, axis=0)


def kernel(inputs, embedding):
    n, dim = inputs.shape
    grid = n // _BLOCK
    emb_t = jnp.transpose(embedding, (0, 2, 1))
    q, idx, loss = pl.pallas_call(
        _rvq_kernel,
        grid=(grid,),
        in_specs=[
            pl.BlockSpec((_BLOCK, dim), lambda i: (i, 0)),
            pl.BlockSpec((_LEVELS, _CODES, _DIM), lambda i: (0, 0, 0)),
            pl.BlockSpec((_LEVELS, _DIM, _CODES), lambda i: (0, 0, 0)),
        ],
        out_specs=(
            pl.BlockSpec((_BLOCK, dim), lambda i: (i, 0)),
            pl.BlockSpec((_LEVELS, _BLOCK), lambda i: (0, i)),
            pl.BlockSpec((_LEVELS, _BLOCK), lambda i: (0, 0)),
        ),
        out_shape=(
            jax.ShapeDtypeStruct((n, dim), jnp.float32),
            jax.ShapeDtypeStruct((_LEVELS, n), jnp.int32),
            jax.ShapeDtypeStruct((_LEVELS, _BLOCK), jnp.float32),
        ),
        scratch_shapes=[
            pltpu.VMEM((_LEVELS, _CODES, _BLOCK), jnp.float32),
            pltpu.VMEM((_CODES, _BLOCK), jnp.int32),
        ],
    )(inputs, embedding, emb_t)
    denom = jnp.float32(n * dim)
    per_level = jnp.sum(loss, axis=1) / denom
    cb = per_level[0] + per_level[1] + per_level[2] + per_level[3]
    commit = cb
    vq = cb + jnp.float32(_COMMIT) * commit
    return (q, idx, vq, cb, commit)
